# Initial kernel scaffold; baseline (speedup 1.0000x reference)
import jax
import jax.numpy as jnp
from jax.experimental import pallas as pl


def kernel(x, edge_index, edge_attr, W, b):
    raise NotImplementedError("write your pallas kernel here")


# speedup vs baseline: 50.0504x; 50.0504x over previous
import functools

import jax
import jax.numpy as jnp
from jax import lax
from jax.experimental import pallas as pl
from jax.experimental.pallas import tpu as pltpu
from jax.experimental.pallas import tpu_sc as plsc

N, E, D = 10000, 320000, 128
NC, NS = 2, 16
NW = NC * NS
L = 16
SLICE = 320
EC = E // NW
_MESH = plsc.VectorSubcoreMesh(core_axis_name="c", subcore_axis_name="s")


@functools.partial(
    pl.kernel,
    out_type=(
        jax.ShapeDtypeStruct((N,), jnp.float32),
        jax.ShapeDtypeStruct((NC, N), jnp.float32),
    ),
    mesh=_MESH,
    compiler_params=pltpu.CompilerParams(needs_layout_passes=False),
    scratch_types=(
        pltpu.VMEM((SLICE * D,), jnp.float32),
        pltpu.VMEM((D,), jnp.float32),
        pltpu.VMEM((L,), jnp.float32),
        pltpu.VMEM((SLICE,), jnp.float32),
        pltpu.VMEM((EC,), jnp.int32),
        pltpu.VMEM((EC,), jnp.float32),
        pltpu.VMEM((N,), jnp.float32),
        pltpu.VMEM_SHARED((N,), jnp.float32),
    ),
)
def _diag_deg_kernel(x_hbm, col_hbm, w_hbm, b_hbm, diag_hbm, degp_hbm,
                     x_v, w_v, b_v, z_v, col_v, ones_v, zeros_v, deg_sh):
    cid = lax.axis_index("c")
    sid = lax.axis_index("s")
    wid = cid * NS + sid
    base = pl.multiple_of(jnp.minimum(wid * SLICE, N - SLICE), 8)

    pltpu.sync_copy(x_hbm.at[pl.ds(base * D, SLICE * D)], x_v)
    pltpu.sync_copy(w_hbm, w_v)
    pltpu.sync_copy(b_hbm, b_v)
    pltpu.sync_copy(col_hbm.at[pl.ds(wid * EC, EC)], col_v)

    iota = jnp.arange(L, dtype=jnp.int32)

    def group_body(g, _):
        fidx = (g * L + iota) * D
        acc = jnp.zeros((L,), jnp.float32)
        for d8 in range(D // L):
            wv = w_v[pl.ds(d8 * L, L)]
            for j in range(L):
                d = d8 * L + j
                xa = plsc.load_gather(x_v, [fidx + d])
                acc = acc + xa * wv[j]
        z_v[pl.ds(g * L, L)] = acc
        return 0

    lax.fori_loop(0, SLICE // L, group_body, 0)

    def sig_body(j, _):
        zv = z_v[pl.ds(j * L, L)] + b_v[...]
        z_v[pl.ds(j * L, L)] = 1.0 / (1.0 + jnp.exp(-zv))
        return 0

    lax.fori_loop(0, SLICE // L, sig_body, 0)
    pltpu.sync_copy(z_v, diag_hbm.at[pl.ds(base, SLICE)])

    def fill_ones(k, _):
        ones_v[pl.ds(k * L, L)] = jnp.full((L,), 1.0, jnp.float32)
        return 0

    lax.fori_loop(0, EC // L, fill_ones, 0)

    @pl.when(sid == 0)
    def _():
        def fill_zeros(k, _):
            zeros_v[pl.ds(k * L, L)] = jnp.zeros((L,), jnp.float32)
            return 0
        lax.fori_loop(0, N // L, fill_zeros, 0)
        pltpu.sync_copy(zeros_v, deg_sh)

    plsc.subcore_barrier()
    pltpu.sync_copy(ones_v, deg_sh.at[col_v], add=True)
    plsc.subcore_barrier()

    @pl.when(sid == 0)
    def _():
        pltpu.sync_copy(deg_sh, degp_hbm.at[cid])


@functools.partial(
    pl.kernel,
    out_type=jax.ShapeDtypeStruct((E,), jnp.float32),
    mesh=_MESH,
    compiler_params=pltpu.CompilerParams(needs_layout_passes=False),
    scratch_types=(
        pltpu.VMEM((N,), jnp.float32),
        pltpu.VMEM((N,), jnp.float32),
        pltpu.VMEM((N,), jnp.float32),
        pltpu.VMEM((EC,), jnp.int32),
        pltpu.VMEM((EC,), jnp.int32),
        pltpu.VMEM((EC,), jnp.float32),
        pltpu.VMEM((EC,), jnp.float32),
    ),
)
def _edge_kernel(row_hbm, col_hbm, ea_hbm, diag_hbm, degp_hbm, out_hbm,
                 diag_v, dinv_v, dp1_v, row_v, col_v, ea_v, out_v):
    cid = lax.axis_index("c")
    sid = lax.axis_index("s")
    wid = cid * NS + sid
    off = wid * EC

    pltpu.sync_copy(diag_hbm, diag_v)
    pltpu.sync_copy(degp_hbm.at[0], dinv_v)
    pltpu.sync_copy(degp_hbm.at[1], dp1_v)
    pltpu.sync_copy(row_hbm.at[pl.ds(off, EC)], row_v)
    pltpu.sync_copy(col_hbm.at[pl.ds(off, EC)], col_v)
    pltpu.sync_copy(ea_hbm.at[pl.ds(off, EC)], ea_v)

    def inv_body(j, _):
        s = pl.ds(j * L, L)
        dinv_v[s] = 1.0 / (dinv_v[s] + dp1_v[s])
        return 0

    lax.fori_loop(0, N // L, inv_body, 0)

    def edge_body(i, _):
        s = pl.ds(i * L, L)
        dv = plsc.load_gather(dinv_v, [row_v[s]])
        gv = plsc.load_gather(diag_v, [col_v[s]])
        out_v[s] = ea_v[s] * dv * gv
        return 0

    lax.fori_loop(0, EC // L, edge_body, 0)
    pltpu.sync_copy(out_v, out_hbm.at[pl.ds(off, EC)])


def kernel(x, edge_index, edge_attr, W, b):
    row = edge_index[0]
    col = edge_index[1]
    w_flat = W.reshape(D)
    x_flat = x.reshape(N * D)
    b_vec = jnp.broadcast_to(b.reshape(1), (L,)).astype(jnp.float32)
    diag, degp = _diag_deg_kernel(x_flat, col, w_flat, b_vec)
    adj_val = _edge_kernel(row, col, edge_attr, diag, degp)
    return (edge_index, adj_val)


# speedup vs baseline: 51.6424x; 1.0318x over previous
import functools

import jax
import jax.numpy as jnp
from jax import lax
from jax.experimental import pallas as pl
from jax.experimental.pallas import tpu as pltpu
from jax.experimental.pallas import tpu_sc as plsc

N, E, D = 10000, 320000, 128
NC, NS = 2, 16
NW = NC * NS
L = 16
SLICE = 320
EC = E // NW
_MESH = plsc.VectorSubcoreMesh(core_axis_name="c", subcore_axis_name="s")


@functools.partial(
    pl.kernel,
    out_type=(
        jax.ShapeDtypeStruct((N,), jnp.float32),
        jax.ShapeDtypeStruct((NC, N), jnp.float32),
    ),
    mesh=_MESH,
    compiler_params=pltpu.CompilerParams(needs_layout_passes=False),
    scratch_types=(
        pltpu.VMEM((SLICE * D,), jnp.float32),
        pltpu.VMEM((D,), jnp.float32),
        pltpu.VMEM((L,), jnp.float32),
        pltpu.VMEM((SLICE,), jnp.float32),
        pltpu.VMEM((EC,), jnp.int32),
        pltpu.VMEM((EC,), jnp.float32),
        pltpu.VMEM((N,), jnp.float32),
        pltpu.VMEM_SHARED((N,), jnp.float32),
    ),
)
def _diag_deg_kernel(x_hbm, col_hbm, w_hbm, b_hbm, diag_hbm, degp_hbm,
                     x_v, w_v, b_v, z_v, col_v, ones_v, zeros_v, deg_sh):
    cid = lax.axis_index("c")
    sid = lax.axis_index("s")
    wid = cid * NS + sid
    base = pl.multiple_of(jnp.minimum(wid * SLICE, N - SLICE), 8)

    pltpu.sync_copy(x_hbm.at[pl.ds(base * D, SLICE * D)], x_v)
    pltpu.sync_copy(w_hbm, w_v)
    pltpu.sync_copy(b_hbm, b_v)
    pltpu.sync_copy(col_hbm.at[pl.ds(wid * EC, EC)], col_v)

    iota = jnp.arange(L, dtype=jnp.int32)

    def group_body(g, _):
        fidx = (g * L + iota) * D
        acc = jnp.zeros((L,), jnp.float32)
        for d8 in range(D // L):
            wv = w_v[pl.ds(d8 * L, L)]
            for j in range(L):
                d = d8 * L + j
                xa = plsc.load_gather(x_v, [fidx + d])
                acc = acc + xa * wv[j]
        z_v[pl.ds(g * L, L)] = acc
        return 0

    lax.fori_loop(0, SLICE // L, group_body, 0)

    def sig_body(j, _):
        zv = z_v[pl.ds(j * L, L)] + b_v[...]
        z_v[pl.ds(j * L, L)] = 1.0 / (1.0 + jnp.exp(-zv))
        return 0

    lax.fori_loop(0, SLICE // L, sig_body, 0, unroll=4)
    pltpu.sync_copy(z_v, diag_hbm.at[pl.ds(base, SLICE)])

    def fill_ones(k, _):
        ones_v[pl.ds(k * L, L)] = jnp.full((L,), 1.0, jnp.float32)
        return 0

    lax.fori_loop(0, EC // L, fill_ones, 0, unroll=8)

    @pl.when(sid == 0)
    def _():
        def fill_zeros(k, _):
            zeros_v[pl.ds(k * L, L)] = jnp.zeros((L,), jnp.float32)
            return 0
        lax.fori_loop(0, N // L, fill_zeros, 0, unroll=8)
        pltpu.sync_copy(zeros_v, deg_sh)

    plsc.subcore_barrier()
    pltpu.sync_copy(ones_v, deg_sh.at[col_v], add=True)
    plsc.subcore_barrier()

    @pl.when(sid == 0)
    def _():
        pltpu.sync_copy(deg_sh, degp_hbm.at[cid])


@functools.partial(
    pl.kernel,
    out_type=jax.ShapeDtypeStruct((E,), jnp.float32),
    mesh=_MESH,
    compiler_params=pltpu.CompilerParams(needs_layout_passes=False),
    scratch_types=(
        pltpu.VMEM((N,), jnp.float32),
        pltpu.VMEM((N,), jnp.float32),
        pltpu.VMEM((N,), jnp.float32),
        pltpu.VMEM((EC,), jnp.int32),
        pltpu.VMEM((EC,), jnp.int32),
        pltpu.VMEM((EC,), jnp.float32),
        pltpu.VMEM((EC,), jnp.float32),
    ),
)
def _edge_kernel(row_hbm, col_hbm, ea_hbm, diag_hbm, degp_hbm, out_hbm,
                 diag_v, dinv_v, dp1_v, row_v, col_v, ea_v, out_v):
    cid = lax.axis_index("c")
    sid = lax.axis_index("s")
    wid = cid * NS + sid
    off = wid * EC

    pltpu.sync_copy(diag_hbm, diag_v)
    pltpu.sync_copy(degp_hbm.at[0], dinv_v)
    pltpu.sync_copy(degp_hbm.at[1], dp1_v)
    pltpu.sync_copy(row_hbm.at[pl.ds(off, EC)], row_v)
    pltpu.sync_copy(col_hbm.at[pl.ds(off, EC)], col_v)
    pltpu.sync_copy(ea_hbm.at[pl.ds(off, EC)], ea_v)

    def inv_body(j, _):
        s = pl.ds(j * L, L)
        dinv_v[s] = 1.0 / (dinv_v[s] + dp1_v[s])
        return 0

    lax.fori_loop(0, N // L, inv_body, 0, unroll=8)

    def edge_body(i, _):
        s = pl.ds(i * L, L)
        dv = plsc.load_gather(dinv_v, [row_v[s]])
        gv = plsc.load_gather(diag_v, [col_v[s]])
        out_v[s] = ea_v[s] * dv * gv
        return 0

    lax.fori_loop(0, EC // L, edge_body, 0, unroll=8)
    pltpu.sync_copy(out_v, out_hbm.at[pl.ds(off, EC)])


def kernel(x, edge_index, edge_attr, W, b):
    row = edge_index[0]
    col = edge_index[1]
    w_flat = W.reshape(D)
    x_flat = x.reshape(N * D)
    b_vec = jnp.broadcast_to(b.reshape(1), (L,)).astype(jnp.float32)
    diag, degp = _diag_deg_kernel(x_flat, col, w_flat, b_vec)
    adj_val = _edge_kernel(row, col, edge_attr, diag, degp)
    return (edge_index, adj_val)


# speedup vs baseline: 60.1274x; 1.1643x over previous
import functools

import jax
import jax.numpy as jnp
from jax import lax
from jax.experimental import pallas as pl
from jax.experimental.pallas import tpu as pltpu
from jax.experimental.pallas import tpu_sc as plsc

N, E, D = 10000, 320000, 128
NC, NS = 2, 16
NW = NC * NS
L = 16
SLICE = 320
NSL = 640
EC = E // NW
ECA = E // NS
_MESH = plsc.VectorSubcoreMesh(core_axis_name="c", subcore_axis_name="s")
_PARAMS = pltpu.CompilerParams(needs_layout_passes=False)


@functools.partial(
    pl.kernel,
    out_type=(
        jax.ShapeDtypeStruct((N,), jnp.float32),
        jax.ShapeDtypeStruct((N,), jnp.float32),
    ),
    mesh=_MESH,
    compiler_params=_PARAMS,
    scratch_types=(
        pltpu.VMEM((SLICE * D,), jnp.float32),
        pltpu.VMEM((D,), jnp.float32),
        pltpu.VMEM((L,), jnp.float32),
        pltpu.VMEM((SLICE,), jnp.float32),
        pltpu.VMEM((ECA,), jnp.int32),
        pltpu.VMEM((ECA,), jnp.float32),
        pltpu.VMEM((NSL,), jnp.float32),
        pltpu.VMEM_SHARED((N,), jnp.float32),
        pltpu.SemaphoreType.DMA,
        pltpu.SemaphoreType.DMA,
    ),
)
def _diag_deg_kernel(x_hbm, col_hbm, w_hbm, b_hbm, diag_hbm, dinv_hbm,
                     x_v, w_v, b_v, z_v, col_v, ones_v, dsl_v, deg_sh,
                     sem_x, sem_h):
    cid = lax.axis_index("c")
    sid = lax.axis_index("s")
    wid = cid * NS + sid
    base = pl.multiple_of(jnp.minimum(wid * SLICE, N - SLICE), 8)
    nbase = pl.multiple_of(jnp.minimum(sid * NSL, N - NSL), 8)

    hx = pltpu.async_copy(x_hbm.at[pl.ds(base * D, SLICE * D)], x_v, sem_x)
    pltpu.sync_copy(col_hbm.at[pl.ds(sid * ECA, ECA)], col_v)
    pltpu.sync_copy(w_hbm, w_v)
    pltpu.sync_copy(b_hbm, b_v)

    def fill_ones(k, _):
        ones_v[pl.ds(k * L, L)] = jnp.full((L,), 1.0, jnp.float32)
        return 0

    lax.fori_loop(0, ECA // L, fill_ones, 0, unroll=8)

    def fill_zero(k, _):
        dsl_v[pl.ds(k * L, L)] = jnp.zeros((L,), jnp.float32)
        return 0

    lax.fori_loop(0, NSL // L, fill_zero, 0, unroll=8)
    pltpu.sync_copy(dsl_v, deg_sh.at[pl.ds(nbase, NSL)])
    plsc.subcore_barrier()

    hh = pltpu.async_copy(ones_v, deg_sh.at[col_v], sem_h, add=True)

    iota = jnp.arange(L, dtype=jnp.int32)
    hx.wait()

    def group_body(g, _):
        fidx = (g * L + iota) * D
        acc = [jnp.zeros((L,), jnp.float32) for _ in range(4)]
        for d8 in range(D // L):
            wv = w_v[pl.ds(d8 * L, L)]
            for j in range(L):
                d = d8 * L + j
                xa = plsc.load_gather(x_v, [fidx + d])
                acc[d % 4] = acc[d % 4] + xa * wv[j]
        z_v[pl.ds(g * L, L)] = (acc[0] + acc[1]) + (acc[2] + acc[3])
        return 0

    lax.fori_loop(0, SLICE // L, group_body, 0)

    def sig_body(j, _):
        zv = z_v[pl.ds(j * L, L)] + b_v[...]
        z_v[pl.ds(j * L, L)] = 1.0 / (1.0 + jnp.exp(-zv))
        return 0

    lax.fori_loop(0, SLICE // L, sig_body, 0, unroll=4)
    pltpu.sync_copy(z_v, diag_hbm.at[pl.ds(base, SLICE)])

    hh.wait()
    plsc.subcore_barrier()

    pltpu.sync_copy(deg_sh.at[pl.ds(nbase, NSL)], dsl_v)

    def inv_body(k, _):
        s = pl.ds(k * L, L)
        dsl_v[s] = 1.0 / dsl_v[s]
        return 0

    lax.fori_loop(0, NSL // L, inv_body, 0, unroll=8)

    @pl.when(cid == 0)
    def _():
        pltpu.sync_copy(dsl_v, dinv_hbm.at[pl.ds(nbase, NSL)])


@functools.partial(
    pl.kernel,
    out_type=jax.ShapeDtypeStruct((E,), jnp.float32),
    mesh=_MESH,
    compiler_params=_PARAMS,
    scratch_types=(
        pltpu.VMEM((N,), jnp.float32),
        pltpu.VMEM((N,), jnp.float32),
        pltpu.VMEM((EC,), jnp.int32),
        pltpu.VMEM((EC,), jnp.int32),
        pltpu.VMEM((EC,), jnp.float32),
        pltpu.VMEM((EC,), jnp.float32),
        pltpu.SemaphoreType.DMA,
    ),
)
def _edge_kernel(row_hbm, col_hbm, ea_hbm, diag_hbm, dinv_hbm, out_hbm,
                 diag_v, dinv_v, row_v, col_v, ea_v, out_v, sem):
    cid = lax.axis_index("c")
    sid = lax.axis_index("s")
    wid = cid * NS + sid
    off = wid * EC

    copies = (
        pltpu.make_async_copy(diag_hbm, diag_v, sem),
        pltpu.make_async_copy(dinv_hbm, dinv_v, sem),
        pltpu.make_async_copy(row_hbm.at[pl.ds(off, EC)], row_v, sem),
        pltpu.make_async_copy(col_hbm.at[pl.ds(off, EC)], col_v, sem),
        pltpu.make_async_copy(ea_hbm.at[pl.ds(off, EC)], ea_v, sem),
    )
    for c in copies:
        c.start()
    for c in copies:
        c.wait()

    def edge_body(i, _):
        s = pl.ds(i * L, L)
        dv = plsc.load_gather(dinv_v, [row_v[s]])
        gv = plsc.load_gather(diag_v, [col_v[s]])
        out_v[s] = ea_v[s] * dv * gv
        return 0

    lax.fori_loop(0, EC // L, edge_body, 0, unroll=8)
    pltpu.sync_copy(out_v, out_hbm.at[pl.ds(off, EC)])


def kernel(x, edge_index, edge_attr, W, b):
    row = edge_index[0]
    col = edge_index[1]
    w_flat = W.reshape(D)
    x_flat = x.reshape(N * D)
    b_vec = jnp.broadcast_to(b.reshape(1), (L,)).astype(jnp.float32)
    diag, dinv = _diag_deg_kernel(x_flat, col, w_flat, b_vec)
    adj_val = _edge_kernel(row, col, edge_attr, diag, dinv)
    return (edge_index, adj_val)
